# baseline (speedup 1.0000x reference)
import jax
import jax.numpy as jnp
from jax import lax
from jax.experimental import pallas as pl
from jax.experimental.pallas import tpu as pltpu
from jax.experimental.pallas import tpu_sc as plsc

N_GENES = 10000
WIDTH = 4
LATENT = 256
FAN_IN = 32
HIDDEN = N_GENES * WIDTH
NNZ1 = HIDDEN * FAN_IN
BATCH = 256
NEG_SLOPE = 0.01

CHUNK = 128
CN = CHUNK * FAN_IN
N_CHUNKS = 316
HIDDEN_PAD = N_CHUNKS * CHUNK
GENES_PAD = HIDDEN_PAD // WIDTH
NUM_WORKERS = 32
ROUNDS = 10
H_B = 4 * CHUNK
GT_B = H_B // WIDTH


def _sc_build(cm_hbm, wm_hbm, wt_hbm, cm_v, wm_v, buf0, buf1, sem0, sem1):
    wid = lax.axis_index("s") * 2 + lax.axis_index("c")
    lane = lax.iota(jnp.int32, 16)
    bufs = (buf0, buf1)
    sems = (sem0, sem1)
    zero16 = jnp.zeros((16,), jnp.float32)

    for t in range(ROUNDS):
        b = t % 2
        buf = bufs[b]
        cid = t * NUM_WORKERS + wid

        @pl.when(cid < N_CHUNKS)
        def _chunk():
            if t >= 2:
                pltpu.make_async_copy(buf, wt_hbm.at[cid - 2 * NUM_WORKERS],
                                      sems[b]).wait()

            def zbody(i, carry):
                for j in range(16):
                    buf[pl.ds(i * 256 + j * 16, 16)] = zero16
                return carry

            lax.fori_loop(0, LATENT * CHUNK // 256, zbody, 0)

            base = cid * CHUNK
            ngroups = jnp.clip((HIDDEN - base) // 16, 0, CHUNK // 16)

            @pl.when(ngroups > 0)
            def _scatter():
                stage = jnp.minimum(cid * CN, NNZ1 - CN)
                pltpu.sync_copy(cm_hbm.at[pl.ds(stage, CN)], cm_v)
                pltpu.sync_copy(wm_hbm.at[pl.ds(stage, CN)], wm_v)
                loc0 = base * FAN_IN - stage

                def gbody(g, carry):
                    node = g * 16 + lane
                    nnz = loc0 + node * FAN_IN
                    for k in range(FAN_IN):
                        c = plsc.load_gather(cm_v, [nnz + k])
                        w = plsc.load_gather(wm_v, [nnz + k])
                        plsc.addupdate_scatter(buf, [c * CHUNK + node], w)
                    return carry

                lax.fori_loop(0, ngroups, gbody, 0)

            pltpu.async_copy(buf, wt_hbm.at[cid], sems[b])

    pltpu.make_async_copy(bufs[0], wt_hbm.at[8 * NUM_WORKERS + wid],
                          sems[0]).wait()
    last1 = jnp.where(9 * NUM_WORKERS + wid < N_CHUNKS,
                      9 * NUM_WORKERS + wid, 7 * NUM_WORKERS + wid)
    pltpu.make_async_copy(bufs[1], wt_hbm.at[last1], sems[1]).wait()


def _tc_body(f_ref, wt_ref, b1_ref, w2_ref, b2_ref, out_ref):
    h = jnp.concatenate(
        [jnp.dot(f_ref[...], wt_ref[q], preferred_element_type=jnp.float32)
         for q in range(4)], axis=1)
    h = h + b1_ref[...]
    h = jnp.where(h >= 0, h, NEG_SLOPE * h)
    hid_iota = lax.broadcasted_iota(jnp.int32, (H_B, GT_B), 0)
    gene_iota = lax.broadcasted_iota(jnp.int32, (H_B, GT_B), 1)
    pool = jnp.where(hid_iota // WIDTH == gene_iota,
                     w2_ref[...].reshape(H_B, 1), 0.0)
    out_ref[...] = jnp.dot(h, pool, preferred_element_type=jnp.float32) + b2_ref[...]


def kernel(features, w1, b1, w2, b2, conn1_row, conn1_col, conn2_row, conn2_col):
    del conn1_row, conn2_row, conn2_col
    pad_h = HIDDEN_PAD - HIDDEN
    b1p = jnp.pad(b1, (0, pad_h)).reshape(1, HIDDEN_PAD)
    w2p = jnp.pad(w2, (0, pad_h)).reshape(1, HIDDEN_PAD)
    b2p = jnp.pad(b2, (0, GENES_PAD - N_GENES)).reshape(1, GENES_PAD)

    mesh = plsc.VectorSubcoreMesh(core_axis_name="c", subcore_axis_name="s")
    wt_flat = pl.kernel(
        _sc_build,
        out_type=jax.ShapeDtypeStruct((N_CHUNKS, LATENT * CHUNK), jnp.float32),
        mesh=mesh,
        scratch_types=[
            pltpu.VMEM((CN,), jnp.int32),
            pltpu.VMEM((CN,), jnp.float32),
            pltpu.VMEM((LATENT * CHUNK,), jnp.float32),
            pltpu.VMEM((LATENT * CHUNK,), jnp.float32),
            pltpu.SemaphoreType.DMA,
            pltpu.SemaphoreType.DMA,
        ],
        compiler_params=pltpu.CompilerParams(needs_layout_passes=False),
    )(conn1_col, w1)
    wt3 = wt_flat.reshape(N_CHUNKS, LATENT, CHUNK)

    out = pl.pallas_call(
        _tc_body,
        grid=(N_CHUNKS // 4,),
        in_specs=[
            pl.BlockSpec((BATCH, LATENT), lambda i: (0, 0)),
            pl.BlockSpec((4, LATENT, CHUNK), lambda i: (i, 0, 0)),
            pl.BlockSpec((1, H_B), lambda i: (0, i)),
            pl.BlockSpec((1, H_B), lambda i: (0, i)),
            pl.BlockSpec((1, GT_B), lambda i: (0, i)),
        ],
        out_specs=pl.BlockSpec((BATCH, GT_B), lambda i: (0, i)),
        out_shape=jax.ShapeDtypeStruct((BATCH, GENES_PAD), jnp.float32),
    )(features, wt3, b1p, w2p, b2p)
    return out[:, :N_GENES]


# speedup vs baseline: 1.6832x; 1.6832x over previous
import jax
import jax.numpy as jnp
from jax import lax
from jax.experimental import pallas as pl
from jax.experimental.pallas import tpu as pltpu
from jax.experimental.pallas import tpu_sc as plsc

N_GENES = 10000
WIDTH = 4
LATENT = 256
FAN_IN = 32
HIDDEN = N_GENES * WIDTH
NNZ1 = HIDDEN * FAN_IN
BATCH = 256
NEG_SLOPE = 0.01

CHUNK = 128
CN = CHUNK * FAN_IN
CW = LATENT * CHUNK
N_CHUNKS = 316
HIDDEN_PAD = N_CHUNKS * CHUNK
NUM_WORKERS = 32
ROUNDS = 10
H_B = 4 * CHUNK
GT_B = H_B // WIDTH


def _sc_build(cm_hbm, wm_hbm, wt_hbm, cm_v, wm_v, buf0, buf1,
              sem0, sem1, sem_c, sem_w):
    wid = lax.axis_index("s") * 2 + lax.axis_index("c")
    lane = lax.iota(jnp.int32, 16)
    bufs = (buf0, buf1)
    sems = (sem0, sem1)
    zero16 = jnp.zeros((16,), jnp.float32)

    for t in range(ROUNDS):
        b = t % 2
        buf = bufs[b]
        cid = t * NUM_WORKERS + wid

        @pl.when(cid < N_CHUNKS)
        def _chunk():
            base = cid * CHUNK
            ngroups = jnp.clip((HIDDEN - base) // 16, 0, CHUNK // 16)
            stage = jnp.minimum(cid * CN, NNZ1 - CN)

            @pl.when(ngroups > 0)
            def _stage():
                pltpu.async_copy(cm_hbm.at[pl.ds(stage, CN)], cm_v, sem_c)
                pltpu.async_copy(wm_hbm.at[pl.ds(stage, CN)], wm_v, sem_w)

            if t >= 2:
                pltpu.make_async_copy(buf, wt_hbm.at[pl.ds((cid - 2 * NUM_WORKERS) * CW, CW)],
                                      sems[b]).wait()

            def zbody(i, carry):
                for j in range(16):
                    buf[pl.ds(i * 256 + j * 16, 16)] = zero16
                return carry

            lax.fori_loop(0, CW // 256, zbody, 0)

            @pl.when(ngroups > 0)
            def _scatter():
                pltpu.make_async_copy(cm_hbm.at[pl.ds(stage, CN)], cm_v,
                                      sem_c).wait()
                pltpu.make_async_copy(wm_hbm.at[pl.ds(stage, CN)], wm_v,
                                      sem_w).wait()
                loc0 = base * FAN_IN - stage

                def gbody(g, carry):
                    node = g * 16 + lane
                    nnz = loc0 + node * FAN_IN
                    for k0 in range(0, FAN_IN, 4):
                        cs = [plsc.load_gather(cm_v, [nnz + (k0 + j)])
                              for j in range(4)]
                        ws = [plsc.load_gather(wm_v, [nnz + (k0 + j)])
                              for j in range(4)]
                        for j in range(4):
                            plsc.addupdate_scatter(
                                buf, [cs[j] * CHUNK + node], ws[j])
                    return carry

                lax.fori_loop(0, ngroups, gbody, 0)

            pltpu.async_copy(buf, wt_hbm.at[pl.ds(cid * CW, CW)], sems[b])

    pltpu.make_async_copy(bufs[0], wt_hbm.at[pl.ds((8 * NUM_WORKERS + wid) * CW, CW)],
                          sems[0]).wait()
    last1 = jnp.where(9 * NUM_WORKERS + wid < N_CHUNKS,
                      9 * NUM_WORKERS + wid, 7 * NUM_WORKERS + wid)
    pltpu.make_async_copy(bufs[1], wt_hbm.at[pl.ds(last1 * CW, CW)],
                          sems[1]).wait()


def _tc_body(f_ref, wt_ref, b1_ref, w2_ref, b2_ref, out_ref):
    i = pl.program_id(0)
    h = jnp.concatenate(
        [jnp.dot(f_ref[...], wt_ref[pl.ds(q * LATENT, LATENT), :],
                 preferred_element_type=jnp.float32)
         for q in range(4)], axis=1)
    h = h + b1_ref[...]
    h = jnp.where(h >= 0, h, NEG_SLOPE * h)
    nvalid = HIDDEN - i * H_B
    col = lax.broadcasted_iota(jnp.int32, (BATCH, H_B), 1)
    h = jnp.where(col < nvalid, h, 0.0)
    hid_iota = lax.broadcasted_iota(jnp.int32, (H_B, GT_B), 0)
    gene_iota = lax.broadcasted_iota(jnp.int32, (H_B, GT_B), 1)
    pool = jnp.where(hid_iota // WIDTH == gene_iota,
                     w2_ref[...].reshape(H_B, 1), 0.0)
    out_ref[...] = jnp.dot(h, pool, preferred_element_type=jnp.float32) + b2_ref[...]


def kernel(features, w1, b1, w2, b2, conn1_row, conn1_col, conn2_row, conn2_col):
    del conn1_row, conn2_row, conn2_col

    mesh = plsc.VectorSubcoreMesh(core_axis_name="c", subcore_axis_name="s")
    wt_flat = pl.kernel(
        _sc_build,
        out_type=jax.ShapeDtypeStruct((N_CHUNKS * CW,), jnp.float32),
        mesh=mesh,
        scratch_types=[
            pltpu.VMEM((CN,), jnp.int32),
            pltpu.VMEM((CN,), jnp.float32),
            pltpu.VMEM((CW,), jnp.float32),
            pltpu.VMEM((CW,), jnp.float32),
            pltpu.SemaphoreType.DMA,
            pltpu.SemaphoreType.DMA,
            pltpu.SemaphoreType.DMA,
            pltpu.SemaphoreType.DMA,
        ],
        compiler_params=pltpu.CompilerParams(needs_layout_passes=False),
    )(conn1_col, w1)
    wt2 = wt_flat.reshape(N_CHUNKS * LATENT, CHUNK)

    out = pl.pallas_call(
        _tc_body,
        grid=(N_CHUNKS // 4,),
        in_specs=[
            pl.BlockSpec((BATCH, LATENT), lambda i: (0, 0)),
            pl.BlockSpec((4 * LATENT, CHUNK), lambda i: (i, 0)),
            pl.BlockSpec((1, H_B), lambda i: (0, i)),
            pl.BlockSpec((1, H_B), lambda i: (0, i)),
            pl.BlockSpec((1, GT_B), lambda i: (0, i)),
        ],
        out_specs=pl.BlockSpec((BATCH, GT_B), lambda i: (0, i)),
        out_shape=jax.ShapeDtypeStruct((BATCH, N_GENES), jnp.float32),
    )(features, wt2, b1.reshape(1, HIDDEN), w2.reshape(1, HIDDEN),
      b2.reshape(1, N_GENES))
    return out


# speedup vs baseline: 1.9827x; 1.1779x over previous
import jax
import jax.numpy as jnp
from jax import lax
from jax.experimental import pallas as pl
from jax.experimental.pallas import tpu as pltpu
from jax.experimental.pallas import tpu_sc as plsc

N_GENES = 10000
WIDTH = 4
LATENT = 256
FAN_IN = 32
HIDDEN = N_GENES * WIDTH
NNZ1 = HIDDEN * FAN_IN
BATCH = 256
NEG_SLOPE = 0.01

CHUNK = 128
CN = CHUNK * FAN_IN
CW = LATENT * CHUNK
N_CHUNKS = 320
HIDDEN_PAD = N_CHUNKS * CHUNK
NUM_WORKERS = 32
ROUNDS = N_CHUNKS // NUM_WORKERS
TC_Q = 8
H_B = TC_Q * CHUNK
GT_B = H_B // WIDTH


def _sc_build(cm_hbm, wm_hbm, wt_hbm, cm_v, wm_v, buf0, buf1,
              sem0, sem1, sem_c, sem_w):
    wid = lax.axis_index("s") * 2 + lax.axis_index("c")
    lane = lax.iota(jnp.int32, 16)
    bufs = (buf0, buf1)
    sems = (sem0, sem1)
    zero16 = jnp.zeros((16,), jnp.float32)

    for t in range(ROUNDS):
        b = t % 2
        buf = bufs[b]
        cid = t * NUM_WORKERS + wid
        base = cid * CHUNK
        ngroups = jnp.clip((HIDDEN - base) // 16, 0, CHUNK // 16)
        stage = jnp.minimum(cid * CN, NNZ1 - CN)

        @pl.when(ngroups > 0)
        def _stage():
            pltpu.async_copy(cm_hbm.at[pl.ds(stage, CN)], cm_v, sem_c)
            pltpu.async_copy(wm_hbm.at[pl.ds(stage, CN)], wm_v, sem_w)

        if t >= 2:
            pltpu.make_async_copy(buf, wt_hbm.at[pl.ds((cid - 2 * NUM_WORKERS) * CW, CW)],
                                  sems[b]).wait()

        def zbody(i, carry):
            for j in range(16):
                buf[pl.ds(i * 256 + j * 16, 16)] = zero16
            return carry

        lax.fori_loop(0, CW // 256, zbody, 0)

        @pl.when(ngroups > 0)
        def _scatter():
            pltpu.make_async_copy(cm_hbm.at[pl.ds(stage, CN)], cm_v,
                                  sem_c).wait()
            pltpu.make_async_copy(wm_hbm.at[pl.ds(stage, CN)], wm_v,
                                  sem_w).wait()
            loc0 = base * FAN_IN - stage

            def gbody(g, carry):
                node = g * 16 + lane
                nnz = loc0 + node * FAN_IN
                for k0 in range(0, FAN_IN, 4):
                    cs = [plsc.load_gather(cm_v, [nnz + (k0 + j)])
                          for j in range(4)]
                    ws = [plsc.load_gather(wm_v, [nnz + (k0 + j)])
                          for j in range(4)]
                    for j in range(4):
                        plsc.addupdate_scatter(
                            buf, [cs[j] * CHUNK + node], ws[j])
                return carry

            lax.fori_loop(0, ngroups, gbody, 0)

        pltpu.async_copy(buf, wt_hbm.at[pl.ds(cid * CW, CW)], sems[b])

    pltpu.make_async_copy(bufs[0], wt_hbm.at[pl.ds((8 * NUM_WORKERS + wid) * CW, CW)],
                          sems[0]).wait()
    pltpu.make_async_copy(bufs[1], wt_hbm.at[pl.ds((9 * NUM_WORKERS + wid) * CW, CW)],
                          sems[1]).wait()


def _tc_body(f_ref, wt_ref, b1_ref, w2_ref, b2_ref, out_ref):
    i = pl.program_id(0)
    h = jnp.concatenate(
        [jnp.dot(f_ref[...], wt_ref[pl.ds(q * LATENT, LATENT), :],
                 preferred_element_type=jnp.float32)
         for q in range(TC_Q)], axis=1)
    h = h + b1_ref[...]
    h = jnp.where(h >= 0, h, NEG_SLOPE * h)
    nvalid = HIDDEN - i * H_B
    col = lax.broadcasted_iota(jnp.int32, (BATCH, H_B), 1)
    h = jnp.where(col < nvalid, h, 0.0)
    hid_iota = lax.broadcasted_iota(jnp.int32, (H_B, GT_B), 0)
    gene_iota = lax.broadcasted_iota(jnp.int32, (H_B, GT_B), 1)
    pool = jnp.where(hid_iota // WIDTH == gene_iota,
                     w2_ref[...].reshape(H_B, 1), 0.0)
    out_ref[...] = jnp.dot(h, pool, preferred_element_type=jnp.float32) + b2_ref[...]


def kernel(features, w1, b1, w2, b2, conn1_row, conn1_col, conn2_row, conn2_col):
    del conn1_row, conn2_row, conn2_col

    mesh = plsc.VectorSubcoreMesh(core_axis_name="c", subcore_axis_name="s")
    wt_flat = pl.kernel(
        _sc_build,
        out_type=jax.ShapeDtypeStruct((N_CHUNKS * CW,), jnp.float32),
        mesh=mesh,
        scratch_types=[
            pltpu.VMEM((CN,), jnp.int32),
            pltpu.VMEM((CN,), jnp.float32),
            pltpu.VMEM((CW,), jnp.float32),
            pltpu.VMEM((CW,), jnp.float32),
            pltpu.SemaphoreType.DMA,
            pltpu.SemaphoreType.DMA,
            pltpu.SemaphoreType.DMA,
            pltpu.SemaphoreType.DMA,
        ],
        compiler_params=pltpu.CompilerParams(needs_layout_passes=False),
    )(conn1_col, w1)
    wt2 = wt_flat.reshape(N_CHUNKS * LATENT, CHUNK)

    out = pl.pallas_call(
        _tc_body,
        grid=(N_CHUNKS // TC_Q,),
        in_specs=[
            pl.BlockSpec((BATCH, LATENT), lambda i: (0, 0)),
            pl.BlockSpec((TC_Q * LATENT, CHUNK), lambda i: (i, 0)),
            pl.BlockSpec((1, H_B), lambda i: (0, i)),
            pl.BlockSpec((1, H_B), lambda i: (0, i)),
            pl.BlockSpec((1, GT_B), lambda i: (0, i)),
        ],
        out_specs=pl.BlockSpec((BATCH, GT_B), lambda i: (0, i)),
        out_shape=jax.ShapeDtypeStruct((BATCH, N_GENES), jnp.float32),
    )(features, wt2, b1.reshape(1, HIDDEN), w2.reshape(1, HIDDEN),
      b2.reshape(1, N_GENES))
    return out


# speedup vs baseline: 2.2130x; 1.1162x over previous
import jax
import jax.numpy as jnp
from jax import lax
from jax.experimental import pallas as pl
from jax.experimental.pallas import tpu as pltpu
from jax.experimental.pallas import tpu_sc as plsc

N_GENES = 10000
WIDTH = 4
LATENT = 256
FAN_IN = 32
HIDDEN = N_GENES * WIDTH
NNZ1 = HIDDEN * FAN_IN
BATCH = 256
NEG_SLOPE = 0.01

CHUNK = 128
CN = CHUNK * FAN_IN
CW = LATENT * CHUNK
N_CHUNKS = 320
NUM_WORKERS = 32
N_HALF = N_CHUNKS // 2
ROUNDS = N_HALF // NUM_WORKERS
TC_Q = 8
H_B = TC_Q * CHUNK
GT_B = H_B // WIDTH
STEPS_HALF = N_HALF // TC_Q


def _make_sc_build(c0):
    def _sc_build(cm_hbm, wm_hbm, wt_hbm, cm_v, wm_v, buf0, buf1,
                  sem0, sem1, sem_c, sem_w):
        wid = lax.axis_index("s") * 2 + lax.axis_index("c")
        lane = lax.iota(jnp.int32, 16)
        bufs = (buf0, buf1)
        sems = (sem0, sem1)
        zero16 = jnp.zeros((16,), jnp.float32)

        for t in range(ROUNDS):
            b = t % 2
            buf = bufs[b]
            lcid = t * NUM_WORKERS + wid
            cid = c0 + lcid
            base = cid * CHUNK
            ngroups = jnp.clip((HIDDEN - base) // 16, 0, CHUNK // 16)
            stage = jnp.minimum(cid * CN, NNZ1 - CN)

            @pl.when(ngroups > 0)
            def _stage():
                pltpu.async_copy(cm_hbm.at[pl.ds(stage, CN)], cm_v, sem_c)
                pltpu.async_copy(wm_hbm.at[pl.ds(stage, CN)], wm_v, sem_w)

            if t >= 2:
                pltpu.make_async_copy(
                    buf, wt_hbm.at[pl.ds((lcid - 2 * NUM_WORKERS) * CW, CW)],
                    sems[b]).wait()

            def zbody(i, carry):
                for j in range(16):
                    buf[pl.ds(i * 256 + j * 16, 16)] = zero16
                return carry

            lax.fori_loop(0, CW // 256, zbody, 0)

            @pl.when(ngroups > 0)
            def _scatter():
                pltpu.make_async_copy(cm_hbm.at[pl.ds(stage, CN)], cm_v,
                                      sem_c).wait()
                pltpu.make_async_copy(wm_hbm.at[pl.ds(stage, CN)], wm_v,
                                      sem_w).wait()
                loc0 = base * FAN_IN - stage

                def gbody(g, carry):
                    node = g * 16 + lane
                    nnz = loc0 + node * FAN_IN
                    for k0 in range(0, FAN_IN, 4):
                        cs = [plsc.load_gather(cm_v, [nnz + (k0 + j)])
                              for j in range(4)]
                        ws = [plsc.load_gather(wm_v, [nnz + (k0 + j)])
                              for j in range(4)]
                        for j in range(4):
                            plsc.addupdate_scatter(
                                buf, [cs[j] * CHUNK + node], ws[j])
                    return carry

                lax.fori_loop(0, ngroups, gbody, 0)

            pltpu.async_copy(buf, wt_hbm.at[pl.ds(lcid * CW, CW)], sems[b])

        pltpu.make_async_copy(
            bufs[(ROUNDS - 2) % 2],
            wt_hbm.at[pl.ds(((ROUNDS - 2) * NUM_WORKERS + wid) * CW, CW)],
            sems[(ROUNDS - 2) % 2]).wait()
        pltpu.make_async_copy(
            bufs[(ROUNDS - 1) % 2],
            wt_hbm.at[pl.ds(((ROUNDS - 1) * NUM_WORKERS + wid) * CW, CW)],
            sems[(ROUNDS - 1) % 2]).wait()

    return _sc_build


def _make_tc_body(step0, with_alias):
    def _tc_body(*refs):
        if with_alias:
            f_ref, wt_ref, b1_ref, w2_ref, b2_ref, _prev, out_ref = refs
        else:
            f_ref, wt_ref, b1_ref, w2_ref, b2_ref, out_ref = refs
        i = pl.program_id(0) + step0
        h = jnp.concatenate(
            [jnp.dot(f_ref[...], wt_ref[pl.ds(q * LATENT, LATENT), :],
                     preferred_element_type=jnp.float32)
             for q in range(TC_Q)], axis=1)
        h = h + b1_ref[...]
        h = jnp.where(h >= 0, h, NEG_SLOPE * h)
        nvalid = HIDDEN - i * H_B
        col = lax.broadcasted_iota(jnp.int32, (BATCH, H_B), 1)
        h = jnp.where(col < nvalid, h, 0.0)
        hid_iota = lax.broadcasted_iota(jnp.int32, (H_B, GT_B), 0)
        gene_iota = lax.broadcasted_iota(jnp.int32, (H_B, GT_B), 1)
        pool = jnp.where(hid_iota // WIDTH == gene_iota,
                         w2_ref[...].reshape(H_B, 1), 0.0)
        out_ref[...] = (jnp.dot(h, pool, preferred_element_type=jnp.float32)
                        + b2_ref[...])

    return _tc_body


def _sc_half(c0, conn1_col, w1):
    mesh = plsc.VectorSubcoreMesh(core_axis_name="c", subcore_axis_name="s")
    wt_flat = pl.kernel(
        _make_sc_build(c0),
        out_type=jax.ShapeDtypeStruct((N_HALF * CW,), jnp.float32),
        mesh=mesh,
        scratch_types=[
            pltpu.VMEM((CN,), jnp.int32),
            pltpu.VMEM((CN,), jnp.float32),
            pltpu.VMEM((CW,), jnp.float32),
            pltpu.VMEM((CW,), jnp.float32),
            pltpu.SemaphoreType.DMA,
            pltpu.SemaphoreType.DMA,
            pltpu.SemaphoreType.DMA,
            pltpu.SemaphoreType.DMA,
        ],
        compiler_params=pltpu.CompilerParams(needs_layout_passes=False),
    )(conn1_col, w1)
    return wt_flat.reshape(N_HALF * LATENT, CHUNK)


def _tc_half(step0, wt2, features, b1r, w2r, b2r, prev_out):
    with_alias = prev_out is not None
    in_specs = [
        pl.BlockSpec((BATCH, LATENT), lambda i: (0, 0)),
        pl.BlockSpec((TC_Q * LATENT, CHUNK), lambda i: (i, 0)),
        pl.BlockSpec((1, H_B), lambda i: (0, i + step0)),
        pl.BlockSpec((1, H_B), lambda i: (0, i + step0)),
        pl.BlockSpec((1, GT_B), lambda i: (0, i + step0)),
    ]
    args = [features, wt2, b1r, w2r, b2r]
    kwargs = {}
    if with_alias:
        in_specs.append(pl.BlockSpec(memory_space=pl.ANY))
        args.append(prev_out)
        kwargs["input_output_aliases"] = {5: 0}
    return pl.pallas_call(
        _make_tc_body(step0, with_alias),
        grid=(STEPS_HALF,),
        in_specs=in_specs,
        out_specs=pl.BlockSpec((BATCH, GT_B), lambda i, s0=step0: (0, i + s0)),
        out_shape=jax.ShapeDtypeStruct((BATCH, N_GENES), jnp.float32),
        **kwargs,
    )(*args)


def kernel(features, w1, b1, w2, b2, conn1_row, conn1_col, conn2_row, conn2_col):
    del conn1_row, conn2_row, conn2_col
    b1r = b1.reshape(1, HIDDEN)
    w2r = w2.reshape(1, HIDDEN)
    b2r = b2.reshape(1, N_GENES)

    wt2_a = _sc_half(0, conn1_col, w1)
    wt2_b = _sc_half(N_HALF, conn1_col, w1)
    out_a = _tc_half(0, wt2_a, features, b1r, w2r, b2r, None)
    out = _tc_half(STEPS_HALF, wt2_b, features, b1r, w2r, b2r, out_a)
    return out
